# baseline (speedup 1.0000x reference)
import functools

import jax
import jax.numpy as jnp
from jax import lax
from jax.experimental import pallas as pl
from jax.experimental.pallas import tpu as pltpu

_NC = 100
_NBUF = 3


def _prep(x, W, b, Wa, s_scr, c_scr):
    n = x.shape[0]
    nr = n - _NC
    x16 = x.astype(jnp.bfloat16)
    s = jnp.dot(x16, W.astype(jnp.bfloat16), preferred_element_type=jnp.float32)
    z = jnp.tanh(jnp.dot(x16, Wa.astype(jnp.bfloat16),
                         preferred_element_type=jnp.float32))
    z16 = z.astype(jnp.bfloat16)
    zc = lax.slice(z16, (nr, 0), (n, z16.shape[1]))
    scores = lax.dot_general(z16, zc, (((1,), (1,)), ((), ())),
                             preferred_element_type=jnp.float32)
    m = jnp.max(scores, axis=-1, keepdims=True)
    e = jnp.exp(scores - m)
    a = (e / jnp.sum(e, axis=-1, keepdims=True)).astype(jnp.bfloat16)
    row = lax.broadcasted_iota(jnp.int32, (n, 1), 0)
    a = jnp.where(row < nr, a, 0)

    s16 = s.astype(jnp.bfloat16)
    sc = lax.slice(s16, (nr, 0), (n, s16.shape[1]))
    c_scr[...] = jnp.dot(a, sc, preferred_element_type=jnp.float32) + b
    cc = lax.dot_general(a, s16, (((0,), (0,)), ((), ())),
                         preferred_element_type=jnp.float32)
    c_scr[pl.ds(nr, _NC), :] = cc + b
    s_scr[...] = s16


def _gcn_kernel(x_ref, W1_ref, b1_ref, Wa1_ref, W2_ref, b2_ref, Wa2_ref,
                adj_ref, o_ref, s1_scr, c1_scr, h_scr, s2_scr, c2_scr,
                abuf, sem, *, bm, nblk):
    i = pl.program_id(0)

    def fetch(t):
        blk = lax.rem(t, nblk)
        slot = lax.rem(t, _NBUF)
        pltpu.make_async_copy(
            adj_ref.at[pl.ds(blk * bm, bm), :], abuf.at[slot], sem.at[slot],
        ).start()

    def wait(t):
        blk = lax.rem(t, nblk)
        slot = lax.rem(t, _NBUF)
        pltpu.make_async_copy(
            adj_ref.at[pl.ds(blk * bm, bm), :], abuf.at[slot], sem.at[slot],
        ).wait()
        return slot

    @pl.when(i == 0)
    def _prep1():
        for t in range(_NBUF - 1):
            fetch(t)
        _prep(x_ref[...], W1_ref[...], b1_ref[...], Wa1_ref[...],
              s1_scr, c1_scr)

    def agg_step(t, s_scr, c_scr, store):
        nxt = t + (_NBUF - 1)

        @pl.when(nxt < 2 * nblk)
        def _():
            fetch(nxt)

        slot = wait(t)
        acc = jnp.dot(abuf[slot].astype(jnp.bfloat16), s_scr[...],
                      preferred_element_type=jnp.float32)
        store(lax.rem(t, nblk), acc + c_scr[pl.ds(lax.rem(t, nblk) * bm, bm), :])

    @pl.when((i >= 1) & (i <= nblk))
    def _agg1():
        def store(blk, v):
            h_scr[pl.ds(blk * bm, bm), :] = jnp.maximum(v, 0.0)
        agg_step(i - 1, s1_scr, c1_scr, store)

    @pl.when(i == nblk + 1)
    def _prep2():
        _prep(h_scr[...], W2_ref[...], b2_ref[...], Wa2_ref[...],
              s2_scr, c2_scr)

    @pl.when(i >= nblk + 2)
    def _agg2():
        def store(blk, v):
            o_ref[...] = v
        agg_step(i - 2, s2_scr, c2_scr, store)


def kernel(x, adj, W1, b1, W2, b2, Wa1, Wa2, n_communities):
    n, d0 = x.shape
    d1 = W1.shape[1]
    d2 = W2.shape[1]
    bm = 200
    nblk = n // bm

    shift = jnp.asarray(n_communities, jnp.float32) - jnp.float32(_NC)
    b2_eff = (b2 + shift).reshape(1, d2)

    zero = lambda i: (0, 0)
    out_idx = lambda i: (jnp.maximum(i - (nblk + 2), 0), 0)

    return pl.pallas_call(
        functools.partial(_gcn_kernel, bm=bm, nblk=nblk),
        grid=(2 * nblk + 2,),
        in_specs=[
            pl.BlockSpec((n, d0), zero),
            pl.BlockSpec((d0, d1), zero),
            pl.BlockSpec((1, d1), zero),
            pl.BlockSpec((d0, Wa1.shape[1]), zero),
            pl.BlockSpec((d1, d2), zero),
            pl.BlockSpec((1, d2), zero),
            pl.BlockSpec((d1, Wa2.shape[1]), zero),
            pl.BlockSpec(memory_space=pl.ANY),
        ],
        out_specs=pl.BlockSpec((bm, d2), out_idx),
        out_shape=jax.ShapeDtypeStruct((n, d2), jnp.float32),
        scratch_shapes=[
            pltpu.VMEM((n, d1), jnp.bfloat16),
            pltpu.VMEM((n, d1), jnp.float32),
            pltpu.VMEM((n, d1), jnp.float32),
            pltpu.VMEM((n, d2), jnp.bfloat16),
            pltpu.VMEM((n, d2), jnp.float32),
            pltpu.VMEM((_NBUF, bm, n), jnp.float32),
            pltpu.SemaphoreType.DMA((_NBUF,)),
        ],
        compiler_params=pltpu.CompilerParams(
            dimension_semantics=("arbitrary",),
        ),
    )(x, W1, b1.reshape(1, d1), Wa1, W2, b2_eff, Wa2, adj)


# speedup vs baseline: 1.0167x; 1.0167x over previous
import functools

import jax
import jax.numpy as jnp
from jax import lax
from jax.experimental import pallas as pl
from jax.experimental.pallas import tpu as pltpu

_NC = 100
_NBUF = 7


def _prep(x, W, b, Wa, s_scr, c_scr):
    n = x.shape[0]
    nr = n - _NC
    x16 = x.astype(jnp.bfloat16)
    s = jnp.dot(x16, W.astype(jnp.bfloat16), preferred_element_type=jnp.float32)
    z = jnp.tanh(jnp.dot(x16, Wa.astype(jnp.bfloat16),
                         preferred_element_type=jnp.float32))
    z16 = z.astype(jnp.bfloat16)
    zc = lax.slice(z16, (nr, 0), (n, z16.shape[1]))
    scores = lax.dot_general(z16, zc, (((1,), (1,)), ((), ())),
                             preferred_element_type=jnp.float32)
    m = jnp.max(scores, axis=-1, keepdims=True)
    e = jnp.exp(scores - m)
    a = (e / jnp.sum(e, axis=-1, keepdims=True)).astype(jnp.bfloat16)
    row = lax.broadcasted_iota(jnp.int32, (n, 1), 0)
    a = jnp.where(row < nr, a, 0)

    s16 = s.astype(jnp.bfloat16)
    sc = lax.slice(s16, (nr, 0), (n, s16.shape[1]))
    c_scr[...] = jnp.dot(a, sc, preferred_element_type=jnp.float32) + b
    cc = lax.dot_general(a, s16, (((0,), (0,)), ((), ())),
                         preferred_element_type=jnp.float32)
    c_scr[pl.ds(nr, _NC), :] = cc + b
    s_scr[...] = s16


def _gcn_kernel(x_ref, W1_ref, b1_ref, Wa1_ref, W2_ref, b2_ref, Wa2_ref,
                adj_ref, o_ref, s1_scr, c1_scr, h_scr, s2_scr, c2_scr,
                abuf, sem, *, bm, nblk):
    i = pl.program_id(0)

    hm = bm // 2

    def _copies(t):
        blk = lax.rem(t, nblk)
        slot = lax.rem(t, _NBUF)
        lo = pltpu.make_async_copy(
            adj_ref.at[pl.ds(blk * bm, hm), :],
            abuf.at[slot, pl.ds(0, hm), :], sem.at[slot, 0])
        hi = pltpu.make_async_copy(
            adj_ref.at[pl.ds(blk * bm + hm, hm), :],
            abuf.at[slot, pl.ds(hm, hm), :], sem.at[slot, 1])
        return slot, lo, hi

    def fetch(t):
        _, lo, hi = _copies(t)
        lo.start()
        hi.start()

    def wait(t):
        slot, lo, hi = _copies(t)
        lo.wait()
        hi.wait()
        return slot

    @pl.when(i == 0)
    def _prep1():
        for t in range(_NBUF - 1):
            fetch(t)
        _prep(x_ref[...], W1_ref[...], b1_ref[...], Wa1_ref[...],
              s1_scr, c1_scr)

    def agg_step(t, s_scr, c_scr, store):
        nxt = t + (_NBUF - 1)

        @pl.when(nxt < 2 * nblk)
        def _():
            fetch(nxt)

        slot = wait(t)
        acc = jnp.dot(abuf[slot].astype(jnp.bfloat16), s_scr[...],
                      preferred_element_type=jnp.float32)
        store(lax.rem(t, nblk), acc + c_scr[pl.ds(lax.rem(t, nblk) * bm, bm), :])

    @pl.when((i >= 1) & (i <= nblk))
    def _agg1():
        def store(blk, v):
            h_scr[pl.ds(blk * bm, bm), :] = jnp.maximum(v, 0.0).astype(jnp.bfloat16)
        agg_step(i - 1, s1_scr, c1_scr, store)

    @pl.when(i == nblk + 1)
    def _prep2():
        _prep(h_scr[...], W2_ref[...], b2_ref[...], Wa2_ref[...],
              s2_scr, c2_scr)

    @pl.when(i >= nblk + 2)
    def _agg2():
        def store(blk, v):
            o_ref[...] = v
        agg_step(i - 2, s2_scr, c2_scr, store)


def kernel(x, adj, W1, b1, W2, b2, Wa1, Wa2, n_communities):
    n, d0 = x.shape
    d1 = W1.shape[1]
    d2 = W2.shape[1]
    bm = 80
    nblk = n // bm

    shift = jnp.asarray(n_communities, jnp.float32) - jnp.float32(_NC)
    b2_eff = (b2 + shift).reshape(1, d2)

    zero = lambda i: (0, 0)
    out_idx = lambda i: (jnp.maximum(i - (nblk + 2), 0), 0)

    return pl.pallas_call(
        functools.partial(_gcn_kernel, bm=bm, nblk=nblk),
        grid=(2 * nblk + 2,),
        in_specs=[
            pl.BlockSpec((n, d0), zero),
            pl.BlockSpec((d0, d1), zero),
            pl.BlockSpec((1, d1), zero),
            pl.BlockSpec((d0, Wa1.shape[1]), zero),
            pl.BlockSpec((d1, d2), zero),
            pl.BlockSpec((1, d2), zero),
            pl.BlockSpec((d1, Wa2.shape[1]), zero),
            pl.BlockSpec(memory_space=pl.ANY),
        ],
        out_specs=pl.BlockSpec((bm, d2), out_idx),
        out_shape=jax.ShapeDtypeStruct((n, d2), jnp.float32),
        scratch_shapes=[
            pltpu.VMEM((n, d1), jnp.bfloat16),
            pltpu.VMEM((n, d1), jnp.float32),
            pltpu.VMEM((n, d1), jnp.bfloat16),
            pltpu.VMEM((n, d2), jnp.bfloat16),
            pltpu.VMEM((n, d2), jnp.float32),
            pltpu.VMEM((_NBUF, bm, n), jnp.float32),
            pltpu.SemaphoreType.DMA((_NBUF, 2)),
        ],
        compiler_params=pltpu.CompilerParams(
            dimension_semantics=("arbitrary",),
        ),
    )(x, W1, b1.reshape(1, d1), Wa1, W2, b2_eff, Wa2, adj)


# speedup vs baseline: 1.0256x; 1.0087x over previous
import functools

import jax
import jax.numpy as jnp
from jax import lax
from jax.experimental import pallas as pl
from jax.experimental.pallas import tpu as pltpu

_NC = 100
_NBUF = 7


def _prep(x, W, b, Wa, s_scr, c_scr):
    n = x.shape[0]
    nr = n - _NC
    x16 = x.astype(jnp.bfloat16)
    s = jnp.dot(x16, W.astype(jnp.bfloat16), preferred_element_type=jnp.float32)
    z = jnp.tanh(jnp.dot(x16, Wa.astype(jnp.bfloat16),
                         preferred_element_type=jnp.float32))
    z16 = z.astype(jnp.bfloat16)
    zc = lax.slice(z16, (nr, 0), (n, z16.shape[1]))
    scores = lax.dot_general(z16, zc, (((1,), (1,)), ((), ())),
                             preferred_element_type=jnp.float32)
    m = jnp.max(scores, axis=-1, keepdims=True)
    e = jnp.exp(scores - m)
    a = (e / jnp.sum(e, axis=-1, keepdims=True)).astype(jnp.bfloat16)
    row = lax.broadcasted_iota(jnp.int32, (n, 1), 0)
    a = jnp.where(row < nr, a, 0)

    s16 = s.astype(jnp.bfloat16)
    sc = lax.slice(s16, (nr, 0), (n, s16.shape[1]))
    c_scr[...] = jnp.dot(a, sc, preferred_element_type=jnp.float32) + b
    cc = lax.dot_general(a, s16, (((0,), (0,)), ((), ())),
                         preferred_element_type=jnp.float32)
    c_scr[pl.ds(nr, _NC), :] = cc + b
    s_scr[...] = s16


def _gcn_kernel(x_ref, W1_ref, b1_ref, Wa1_ref, W2_ref, b2_ref, Wa2_ref,
                adj_ref, o_ref, s1_scr, c1_scr, h_scr, s2_scr, c2_scr,
                abuf, sem, *, bm, nblk):
    i = pl.program_id(0)

    def fetch(t):
        blk = lax.rem(t, nblk)
        slot = lax.rem(t, _NBUF)
        pltpu.make_async_copy(
            adj_ref.at[pl.ds(blk * bm, bm), :], abuf.at[slot], sem.at[slot],
        ).start()

    def wait(t):
        blk = lax.rem(t, nblk)
        slot = lax.rem(t, _NBUF)
        pltpu.make_async_copy(
            adj_ref.at[pl.ds(blk * bm, bm), :], abuf.at[slot], sem.at[slot],
        ).wait()
        return slot

    @pl.when(i == 0)
    def _prep1():
        for t in range(_NBUF - 1):
            fetch(t)
        _prep(x_ref[...], W1_ref[...], b1_ref[...], Wa1_ref[...],
              s1_scr, c1_scr)

    def agg_step(t, s_scr, c_scr, store):
        nxt = t + (_NBUF - 1)

        @pl.when(nxt < 2 * nblk)
        def _():
            fetch(nxt)

        slot = wait(t)
        acc = jnp.dot(abuf[slot].astype(jnp.bfloat16), s_scr[...],
                      preferred_element_type=jnp.float32)
        store(lax.rem(t, nblk), acc + c_scr[pl.ds(lax.rem(t, nblk) * bm, bm), :])

    @pl.when((i >= 1) & (i <= nblk))
    def _agg1():
        def store(blk, v):
            h_scr[pl.ds(blk * bm, bm), :] = jnp.maximum(v, 0.0).astype(jnp.bfloat16)
        agg_step(i - 1, s1_scr, c1_scr, store)

    @pl.when(i == nblk + 1)
    def _prep2():
        _prep(h_scr[...], W2_ref[...], b2_ref[...], Wa2_ref[...],
              s2_scr, c2_scr)

    @pl.when(i >= nblk + 2)
    def _agg2():
        def store(blk, v):
            o_ref[...] = v
        agg_step(i - 2, s2_scr, c2_scr, store)


def kernel(x, adj, W1, b1, W2, b2, Wa1, Wa2, n_communities):
    n, d0 = x.shape
    d1 = W1.shape[1]
    d2 = W2.shape[1]
    bm = 80
    nblk = n // bm

    shift = jnp.asarray(n_communities, jnp.float32) - jnp.float32(_NC)
    b2_eff = (b2 + shift).reshape(1, d2)

    zero = lambda i: (0, 0)
    out_idx = lambda i: (jnp.maximum(i - (nblk + 2), 0), 0)

    return pl.pallas_call(
        functools.partial(_gcn_kernel, bm=bm, nblk=nblk),
        grid=(2 * nblk + 2,),
        in_specs=[
            pl.BlockSpec((n, d0), zero),
            pl.BlockSpec((d0, d1), zero),
            pl.BlockSpec((1, d1), zero),
            pl.BlockSpec((d0, Wa1.shape[1]), zero),
            pl.BlockSpec((d1, d2), zero),
            pl.BlockSpec((1, d2), zero),
            pl.BlockSpec((d1, Wa2.shape[1]), zero),
            pl.BlockSpec(memory_space=pl.ANY),
        ],
        out_specs=pl.BlockSpec((bm, d2), out_idx),
        out_shape=jax.ShapeDtypeStruct((n, d2), jnp.float32),
        scratch_shapes=[
            pltpu.VMEM((n, d1), jnp.bfloat16),
            pltpu.VMEM((n, d1), jnp.float32),
            pltpu.VMEM((n, d1), jnp.bfloat16),
            pltpu.VMEM((n, d2), jnp.bfloat16),
            pltpu.VMEM((n, d2), jnp.float32),
            pltpu.VMEM((_NBUF, bm, n), jnp.float32),
            pltpu.SemaphoreType.DMA((_NBUF,)),
        ],
        compiler_params=pltpu.CompilerParams(
            dimension_semantics=("arbitrary",),
        ),
    )(x, W1, b1.reshape(1, d1), Wa1, W2, b2_eff, Wa2, adj)
